# baseline (speedup 1.0000x reference)
import functools

import jax
import jax.numpy as jnp
from jax import lax
from jax.experimental import pallas as pl
from jax.experimental.pallas import tpu as pltpu
from jax.experimental.pallas import tpu_sc as plsc

NC, NS, LANES = 2, 16, 16
NW = NC * NS
CHUNK = 64


def _make_sc_gather(tot, enc, seq_len, code_length, code_number, h, shared_row):
    per_w = tot // NW
    n_chunks = per_w // CHUNK
    assert per_w % CHUNK == 0
    assert enc % CHUNK == 0

    mesh = plsc.VectorSubcoreMesh(core_axis_name="c", subcore_axis_name="s")

    @functools.partial(
        pl.kernel,
        mesh=mesh,
        out_type=jax.ShapeDtypeStruct((tot, h), jnp.float32),
        scratch_types=[
            pltpu.VMEM((CHUNK,), jnp.int32),
            pltpu.VMEM((CHUNK,), jnp.int32),
            pltpu.VMEM((CHUNK,), jnp.int32),
            pltpu.VMEM((CHUNK, h), jnp.float32),
            pltpu.SemaphoreType.DMA,
        ],
    )
    def sc_gather(ids_hbm, mask_hbm, table_hbm, out_hbm,
                  ids_v, mask_v, idx_v, rows_v, sem):
        wid = lax.axis_index("s") * NC + lax.axis_index("c")
        base_w = wid * per_w

        def chunk_body(g, carry):
            base = pl.multiple_of(base_w + g * CHUNK, CHUNK)
            is_enc = base < enc

            @pl.when(is_enc)
            def _():
                pltpu.sync_copy(ids_hbm.at[pl.ds(base, CHUNK)], ids_v)
                pltpu.sync_copy(mask_hbm.at[pl.ds(base, CHUNK)], mask_v)
                for j in range(CHUNK // LANES):
                    p = base + j * LANES + lax.iota(jnp.int32, LANES)
                    pos = (p % seq_len) % code_length
                    idv = ids_v[pl.ds(j * LANES, LANES)]
                    idv = jnp.where(idv == -1, 0, idv)
                    m = mask_v[pl.ds(j * LANES, LANES)]
                    idx = jnp.where(m != 0, pos * code_number + idv, shared_row)
                    idx_v[pl.ds(j * LANES, LANES)] = idx

            @pl.when(jnp.logical_not(is_enc))
            def _():
                for j in range(CHUNK // LANES):
                    p = (base - enc) + j * LANES + lax.iota(jnp.int32, LANES)
                    pos = p % code_length
                    idx = jnp.where(pos == 0, shared_row, (pos - 1) * code_number)
                    idx_v[pl.ds(j * LANES, LANES)] = idx

            pltpu.async_copy(table_hbm.at[idx_v], rows_v, sem).wait()
            pltpu.sync_copy(rows_v, out_hbm.at[pl.ds(base, CHUNK)])
            return carry

        lax.fori_loop(0, n_chunks, chunk_body, 0)

    return sc_gather


def kernel(input_ids, attention_mask, token_tables, shared):
    bsz, seq_len = input_ids.shape
    code_length, code_number, h = token_tables.shape
    enc = bsz * seq_len
    dec = bsz * code_length
    tot = enc + dec

    ids = input_ids.reshape(-1).astype(jnp.int32)
    mask = attention_mask.reshape(-1).astype(jnp.int32)
    shared_row = code_length * code_number
    table = jnp.concatenate(
        [token_tables.reshape(shared_row, h), shared[:1]], axis=0)

    gather = _make_sc_gather(tot, enc, seq_len, code_length, code_number, h,
                             shared_row)
    out = gather(ids, mask, table)
    inputs_embeds = out[:enc].reshape(bsz, seq_len, h)
    decoder_inputs_embeds = out[enc:].reshape(bsz, code_length, h)
    return inputs_embeds, decoder_inputs_embeds
